# baseline (speedup 1.0000x reference)
import functools

import jax
import jax.numpy as jnp
from jax import lax
from jax.experimental import pallas as pl
from jax.experimental.pallas import tpu as pltpu
from jax.experimental.pallas import tpu_sc as plsc

_D = 64
_CHUNK = 384
_STRIDE = 385
_OUTC = 128


@functools.lru_cache(maxsize=None)
def _make_seg_sum(n_rows: int, n_seg: int):
    info = plsc.get_sparse_core_info()
    nw = info.num_cores * info.num_subcores
    segs_w = n_seg // nw
    assert segs_w * nw == n_seg, (n_seg, nw)
    mesh = plsc.VectorSubcoreMesh(core_axis_name="c", subcore_axis_name="s")

    @functools.partial(
        pl.kernel,
        mesh=mesh,
        compiler_params=pltpu.CompilerParams(needs_layout_passes=False),
        out_type=jax.ShapeDtypeStruct((n_seg, _D), jnp.float32),
        scratch_types=[
            pltpu.VMEM((segs_w + 16,), jnp.int32),
            pltpu.VMEM((_D, _STRIDE), jnp.float32),
            pltpu.VMEM((_D, _STRIDE), jnp.float32),
            pltpu.VMEM((_OUTC, _D), jnp.float32),
            pltpu.SemaphoreType.DMA,
            pltpu.SemaphoreType.DMA,
        ],
    )
    def seg_sum(embt_hbm, indptr_hbm, seg_hbm, idx_v, buf_a, buf_b, outbuf,
                sem_a, sem_b):
        wid = lax.axis_index("s") * info.num_cores + lax.axis_index("c")
        seg0 = pl.multiple_of(wid * segs_w, 8)
        pltpu.async_copy(
            indptr_hbm.at[pl.ds(seg0, segs_w + 16)], idx_v, sem_a).wait()

        def ip(i):
            return idx_v[pl.ds(i, 16)][0]

        zeros = jnp.zeros((16,), jnp.float32)
        rowi = lax.iota(jnp.int32, 16)

        def chunk_base(abase, k):
            return pl.multiple_of(
                jnp.minimum(abase + k * _CHUNK,
                            jnp.int32(n_rows - _CHUNK)), 128)

        def start_fetch(abase, k, buf, sem):
            pltpu.async_copy(
                embt_hbm.at[:, pl.ds(chunk_base(abase, k), _CHUNK)],
                buf.at[:, pl.ds(0, _CHUNK)], sem)

        def wait_fetch(buf, sem):
            pltpu.make_async_copy(
                embt_hbm.at[:, pl.ds(0, _CHUNK)],
                buf.at[:, pl.ds(0, _CHUNK)], sem).wait()

        def process(k, buf, st, abase, pend, s_lo):
            seg, p, a0, a1, a2, a3 = st
            base = chunk_base(abase, k)
            hi = jnp.minimum(abase + (k + 1) * _CHUNK, pend)

            def span_add(lo, up, accs):
                @plsc.parallel_loop(lo, up, unroll=4, carry=accs)
                def out(r, accs):
                    cv = jnp.full((16,), r - base, jnp.int32)
                    return (
                        accs[0] + plsc.load_gather(buf, [rowi, cv]),
                        accs[1] + plsc.load_gather(buf, [rowi + 16, cv]),
                        accs[2] + plsc.load_gather(buf, [rowi + 32, cv]),
                        accs[3] + plsc.load_gather(buf, [rowi + 48, cv]),
                    )

                return out

            def cnt_body(b, acc):
                vals = idx_v[pl.ds(s_lo + 1 + b * 16, 16)]
                return acc + plsc.all_reduce_population_count(vals <= hi)[0]

            cnt = lax.fori_loop(0, _OUTC // 16, cnt_body, jnp.int32(0))
            nb = cnt - (seg - s_lo)

            def close_body(j, st):
                seg, p, a0, a1, a2, a3 = st
                e = ip(seg + 1)
                a0, a1, a2, a3 = span_add(p, e, (a0, a1, a2, a3))
                lseg = seg - s_lo
                outbuf[lseg, pl.ds(0, 16)] = a0
                outbuf[lseg, pl.ds(16, 16)] = a1
                outbuf[lseg, pl.ds(32, 16)] = a2
                outbuf[lseg, pl.ds(48, 16)] = a3
                return (seg + 1, e, zeros, zeros, zeros, zeros)

            st = lax.fori_loop(0, nb, close_body, (seg, p, a0, a1, a2, a3))
            seg, p, a0, a1, a2, a3 = st
            a0, a1, a2, a3 = span_add(p, hi, (a0, a1, a2, a3))
            return (seg, hi, a0, a1, a2, a3)

        def tile_body(f, _):
            s_lo = f * _OUTC
            p0 = ip(s_lo)
            pend = ip(s_lo + _OUTC)
            abase = p0 & jnp.int32(~127)
            n_chunks = lax.div(pend - abase, jnp.int32(_CHUNK)) + 1
            n_pairs = lax.div(n_chunks + 1, jnp.int32(2))

            start_fetch(abase, jnp.int32(0), buf_a, sem_a)

            def pair_body(m, st):
                start_fetch(abase, 2 * m + 1, buf_b, sem_b)
                wait_fetch(buf_a, sem_a)
                st = process(2 * m, buf_a, st, abase, pend, s_lo)
                start_fetch(abase, 2 * m + 2, buf_a, sem_a)
                wait_fetch(buf_b, sem_b)
                st = process(2 * m + 1, buf_b, st, abase, pend, s_lo)
                return st

            lax.fori_loop(0, n_pairs, pair_body,
                          (s_lo, p0, zeros, zeros, zeros, zeros))
            wait_fetch(buf_a, sem_a)
            pltpu.async_copy(
                outbuf, seg_hbm.at[pl.ds(seg0 + s_lo, _OUTC)], sem_a).wait()
            return jnp.int32(0)

        lax.fori_loop(0, segs_w // _OUTC, tile_body, jnp.int32(0))

    return seg_sum


@functools.lru_cache(maxsize=None)
def _make_norm(n_clusters: int, n_variants: int):
    vb = 256
    assert n_variants % vb == 0

    def body(lib_ref, seg_ref, out_ref):
        libn = lib_ref[...].reshape(n_clusters, 1, 1) / jnp.float32(1e6)
        v = seg_ref[...] / libn
        mean = jnp.mean(v, axis=0, keepdims=True)
        dlt = v - mean
        var = jnp.sum(dlt * dlt, axis=0, keepdims=True) * (
            1.0 / (n_clusters - 1))
        rel = dlt / (jnp.sqrt(var) + jnp.float32(1e-5))
        out_ref[...] = jnp.concatenate([v, rel], axis=-1)

    return pl.pallas_call(
        body,
        grid=(n_variants // vb,),
        in_specs=[
            pl.BlockSpec((n_clusters, 1), lambda i: (0, 0)),
            pl.BlockSpec((n_clusters, vb, _D), lambda i: (0, i, 0)),
        ],
        out_specs=pl.BlockSpec((n_clusters, vb, 2 * _D), lambda i: (0, i, 0)),
        out_shape=jax.ShapeDtypeStruct(
            (n_clusters, n_variants, 2 * _D), jnp.float32),
    )


def kernel(cut_embedding, local_clusterxvariant_indptr, n_variants,
           n_clusters, cluster_cut_lib):
    nc = cluster_cut_lib.shape[0]
    n_seg = local_clusterxvariant_indptr.shape[0] - 1
    nv = n_seg // nc
    n_rows = cut_embedding.shape[0]

    indptr = local_clusterxvariant_indptr.astype(jnp.int32)
    indptr = jnp.concatenate(
        [indptr, jnp.broadcast_to(indptr[-1], (16,))])

    seg = _make_seg_sum(n_rows, n_seg)(cut_embedding.T, indptr)
    segr = seg.reshape(nc, nv, _D)
    lib2 = cluster_cut_lib.astype(jnp.float32).reshape(nc, 1)
    return _make_norm(nc, nv)(lib2, segr)


# speedup vs baseline: 1.5476x; 1.5476x over previous
import functools

import jax
import jax.numpy as jnp
from jax import lax
from jax.experimental import pallas as pl
from jax.experimental.pallas import tpu as pltpu
from jax.experimental.pallas import tpu_sc as plsc

_D = 64
_CHUNK = 128
_OUTC = 256


@functools.lru_cache(maxsize=None)
def _make_seg_sum(n_rows: int, n_seg: int):
    info = plsc.get_sparse_core_info()
    nw = info.num_cores * info.num_subcores
    segs_w = n_seg // nw
    assert segs_w * nw == n_seg, (n_seg, nw)
    mesh = plsc.VectorSubcoreMesh(core_axis_name="c", subcore_axis_name="s")

    @functools.partial(
        pl.kernel,
        mesh=mesh,
        compiler_params=pltpu.CompilerParams(needs_layout_passes=False),
        out_type=jax.ShapeDtypeStruct((n_seg, _D), jnp.float32),
        scratch_types=[
            pltpu.VMEM((segs_w + 16,), jnp.int32),
            pltpu.VMEM((_CHUNK, 2 * _D), jnp.float32),
            pltpu.VMEM((_CHUNK, 2 * _D), jnp.float32),
            pltpu.VMEM((_OUTC, _D), jnp.float32),
            pltpu.SemaphoreType.DMA,
            pltpu.SemaphoreType.DMA,
        ],
    )
    def seg_sum(emb_hbm, indptr_hbm, seg_hbm, idx_v, buf_a, buf_b, outbuf,
                sem_a, sem_b):
        wid = lax.axis_index("s") * info.num_cores + lax.axis_index("c")
        seg0 = pl.multiple_of(wid * segs_w, 8)
        pltpu.async_copy(
            indptr_hbm.at[pl.ds(seg0, segs_w + 16)], idx_v, sem_a).wait()

        def ip(i):
            return idx_v[pl.ds(i, 16)][0]

        zeros = jnp.zeros((16,), jnp.float32)

        def chunk_base(abase, k):
            return pl.multiple_of(
                jnp.minimum(abase + k * _CHUNK,
                            jnp.int32(n_rows // 2 - _CHUNK)), 8)

        def start_fetch(abase, k, buf, sem):
            pltpu.async_copy(
                emb_hbm.at[pl.ds(chunk_base(abase, k), _CHUNK)], buf, sem)

        def wait_fetch(buf, sem):
            pltpu.make_async_copy(
                emb_hbm.at[pl.ds(0, _CHUNK)], buf, sem).wait()

        def process(k, buf, st, abase, pend, s_lo):
            seg, p, a0, a1, a2, a3 = st
            base = chunk_base(abase, k)
            hi = jnp.minimum(2 * (abase + (k + 1) * _CHUNK), pend)

            def half_add(accs, off, half, m):
                off = jnp.clip(off, jnp.int32(0), jnp.int32(_CHUNK - 1))
                return tuple(
                    a + jnp.where(m, buf[off, pl.ds(half + 16 * j, 16)],
                                  zeros)
                    for j, a in enumerate(accs))

            def span_add(lo, up, accs):
                accs = half_add(accs, (lo >> 1) - base, _D,
                                ((lo & 1) == 1) & (lo < up))

                @plsc.parallel_loop((lo + 1) >> 1, up >> 1, unroll=2,
                                    carry=tuple(accs))
                def out(b, accs):
                    off = b - base
                    return (
                        accs[0] + buf[off, pl.ds(0, 16)]
                        + buf[off, pl.ds(64, 16)],
                        accs[1] + buf[off, pl.ds(16, 16)]
                        + buf[off, pl.ds(80, 16)],
                        accs[2] + buf[off, pl.ds(32, 16)]
                        + buf[off, pl.ds(96, 16)],
                        accs[3] + buf[off, pl.ds(48, 16)]
                        + buf[off, pl.ds(112, 16)],
                    )

                return half_add(out, (up >> 1) - base, 0,
                                ((up & 1) == 1) & (lo < up))

            def cnt_body(b, acc):
                vals = idx_v[pl.ds(s_lo + 1 + b * 16, 16)]
                return acc + plsc.all_reduce_population_count(vals <= hi)[0]

            cnt = lax.fori_loop(0, _OUTC // 16, cnt_body, jnp.int32(0))
            nb = cnt - (seg - s_lo)

            def close_body(j, st):
                seg, p, a0, a1, a2, a3 = st
                e = ip(seg + 1)
                a0, a1, a2, a3 = span_add(p, e, (a0, a1, a2, a3))
                lseg = seg - s_lo
                outbuf[lseg, pl.ds(0, 16)] = a0
                outbuf[lseg, pl.ds(16, 16)] = a1
                outbuf[lseg, pl.ds(32, 16)] = a2
                outbuf[lseg, pl.ds(48, 16)] = a3
                return (seg + 1, e, zeros, zeros, zeros, zeros)

            st = lax.fori_loop(0, nb, close_body, (seg, p, a0, a1, a2, a3))
            seg, p, a0, a1, a2, a3 = st
            a0, a1, a2, a3 = span_add(p, hi, (a0, a1, a2, a3))
            return (seg, hi, a0, a1, a2, a3)

        def tile_body(f, _):
            s_lo = f * _OUTC
            p0 = ip(s_lo)
            pend = ip(s_lo + _OUTC)
            abase = (p0 >> 1) & jnp.int32(~7)
            n_chunks = lax.div(((pend + 1) >> 1) - abase,
                               jnp.int32(_CHUNK)) + 1
            n_pairs = lax.div(n_chunks + 1, jnp.int32(2))

            start_fetch(abase, jnp.int32(0), buf_a, sem_a)

            def pair_body(m, st):
                start_fetch(abase, 2 * m + 1, buf_b, sem_b)
                wait_fetch(buf_a, sem_a)
                st = process(2 * m, buf_a, st, abase, pend, s_lo)
                start_fetch(abase, 2 * m + 2, buf_a, sem_a)
                wait_fetch(buf_b, sem_b)
                st = process(2 * m + 1, buf_b, st, abase, pend, s_lo)
                return st

            lax.fori_loop(0, n_pairs, pair_body,
                          (s_lo, p0, zeros, zeros, zeros, zeros))
            wait_fetch(buf_a, sem_a)
            pltpu.async_copy(
                outbuf, seg_hbm.at[pl.ds(seg0 + s_lo, _OUTC)], sem_a).wait()
            return jnp.int32(0)

        lax.fori_loop(0, segs_w // _OUTC, tile_body, jnp.int32(0))

    return seg_sum


@functools.lru_cache(maxsize=None)
def _make_norm(n_clusters: int, n_variants: int):
    vb = 256
    assert n_variants % vb == 0

    def body(lib_ref, seg_ref, out_ref):
        libn = lib_ref[...].reshape(n_clusters, 1, 1) / jnp.float32(1e6)
        v = seg_ref[...] / libn
        mean = jnp.mean(v, axis=0, keepdims=True)
        dlt = v - mean
        var = jnp.sum(dlt * dlt, axis=0, keepdims=True) * (
            1.0 / (n_clusters - 1))
        rel = dlt / (jnp.sqrt(var) + jnp.float32(1e-5))
        out_ref[...] = jnp.concatenate([v, rel], axis=-1)

    return pl.pallas_call(
        body,
        grid=(n_variants // vb,),
        in_specs=[
            pl.BlockSpec((n_clusters, 1), lambda i: (0, 0)),
            pl.BlockSpec((n_clusters, vb, _D), lambda i: (0, i, 0)),
        ],
        out_specs=pl.BlockSpec((n_clusters, vb, 2 * _D), lambda i: (0, i, 0)),
        out_shape=jax.ShapeDtypeStruct(
            (n_clusters, n_variants, 2 * _D), jnp.float32),
    )


def kernel(cut_embedding, local_clusterxvariant_indptr, n_variants,
           n_clusters, cluster_cut_lib):
    nc = cluster_cut_lib.shape[0]
    n_seg = local_clusterxvariant_indptr.shape[0] - 1
    nv = n_seg // nc
    n_rows = cut_embedding.shape[0]

    indptr = local_clusterxvariant_indptr.astype(jnp.int32)
    indptr = jnp.concatenate(
        [indptr, jnp.broadcast_to(indptr[-1], (16,))])

    emb2 = cut_embedding.reshape(n_rows // 2, 2 * _D)
    seg = _make_seg_sum(n_rows, n_seg)(emb2, indptr)
    segr = seg.reshape(nc, nv, _D)
    lib2 = cluster_cut_lib.astype(jnp.float32).reshape(nc, 1)
    return _make_norm(nc, nv)(lib2, segr)


# speedup vs baseline: 1.8275x; 1.1809x over previous
import functools

import jax
import jax.numpy as jnp
from jax import lax
from jax.experimental import pallas as pl
from jax.experimental.pallas import tpu as pltpu
from jax.experimental.pallas import tpu_sc as plsc

_D = 64
_CHUNK = 384
_OUTC = 128


@functools.lru_cache(maxsize=None)
def _make_seg_sum(n_rows: int, n_seg: int):
    info = plsc.get_sparse_core_info()
    nw = info.num_cores * info.num_subcores
    segs_w = n_seg // nw
    assert segs_w * nw == n_seg, (n_seg, nw)
    mesh = plsc.VectorSubcoreMesh(core_axis_name="c", subcore_axis_name="s")

    @functools.partial(
        pl.kernel,
        mesh=mesh,
        compiler_params=pltpu.CompilerParams(needs_layout_passes=False),
        out_type=jax.ShapeDtypeStruct((n_seg, _D), jnp.float32),
        scratch_types=[
            pltpu.VMEM((segs_w + 16,), jnp.int32),
            pltpu.VMEM((_CHUNK, _D), jnp.float32),
            pltpu.VMEM((_CHUNK, _D), jnp.float32),
            pltpu.VMEM((_OUTC, _D), jnp.float32),
            pltpu.SemaphoreType.DMA,
            pltpu.SemaphoreType.DMA,
        ],
    )
    def seg_sum(emb_hbm, indptr_hbm, seg_hbm, idx_v, buf_a, buf_b, outbuf,
                sem_a, sem_b):
        wid = lax.axis_index("s") * info.num_cores + lax.axis_index("c")
        seg0 = pl.multiple_of(wid * segs_w, 8)
        pltpu.async_copy(
            indptr_hbm.at[pl.ds(seg0, segs_w + 16)], idx_v, sem_a).wait()

        def ip(i):
            return idx_v[pl.ds(i, 16)][0]

        zeros = jnp.zeros((16,), jnp.float32)

        def chunk_base(abase, k):
            return pl.multiple_of(
                jnp.minimum(abase + k * _CHUNK,
                            jnp.int32(n_rows - _CHUNK)), 8)

        def start_fetch(abase, k, buf, sem):
            pltpu.async_copy(
                emb_hbm.at[pl.ds(chunk_base(abase, k), _CHUNK)], buf, sem)

        def wait_fetch(buf, sem):
            pltpu.make_async_copy(
                emb_hbm.at[pl.ds(0, _CHUNK)], buf, sem).wait()

        def process(k, buf, st, abase, pend, s_lo):
            seg, p, a0, a1, a2, a3 = st
            base = chunk_base(abase, k)
            hi = jnp.minimum(abase + (k + 1) * _CHUNK, pend)

            def span_add(lo, up, accs):
                @plsc.parallel_loop(lo, up, unroll=4, carry=accs)
                def out(r, accs):
                    off = r - base
                    return (accs[0] + buf[off, pl.ds(0, 16)],
                            accs[1] + buf[off, pl.ds(16, 16)],
                            accs[2] + buf[off, pl.ds(32, 16)],
                            accs[3] + buf[off, pl.ds(48, 16)])

                return out

            def cnt_body(b, acc):
                vals = idx_v[pl.ds(s_lo + 1 + b * 16, 16)]
                return acc + plsc.all_reduce_population_count(vals <= hi)[0]

            cnt = lax.fori_loop(0, _OUTC // 16, cnt_body, jnp.int32(0))
            nb = cnt - (seg - s_lo)

            def close_body(j, st):
                seg, p, a0, a1, a2, a3 = st
                e = ip(seg + 1)
                a0, a1, a2, a3 = span_add(p, e, (a0, a1, a2, a3))
                lseg = seg - s_lo
                outbuf[lseg, pl.ds(0, 16)] = a0
                outbuf[lseg, pl.ds(16, 16)] = a1
                outbuf[lseg, pl.ds(32, 16)] = a2
                outbuf[lseg, pl.ds(48, 16)] = a3
                return (seg + 1, e, zeros, zeros, zeros, zeros)

            st = lax.fori_loop(0, nb, close_body, (seg, p, a0, a1, a2, a3))
            seg, p, a0, a1, a2, a3 = st
            a0, a1, a2, a3 = span_add(p, hi, (a0, a1, a2, a3))
            return (seg, hi, a0, a1, a2, a3)

        def tile_body(f, _):
            s_lo = f * _OUTC
            p0 = ip(s_lo)
            pend = ip(s_lo + _OUTC)
            abase = p0 & jnp.int32(~7)
            n_chunks = lax.div(pend - abase, jnp.int32(_CHUNK)) + 1
            n_pairs = lax.div(n_chunks + 1, jnp.int32(2))

            start_fetch(abase, jnp.int32(0), buf_a, sem_a)

            def pair_body(m, st):
                start_fetch(abase, 2 * m + 1, buf_b, sem_b)
                wait_fetch(buf_a, sem_a)
                st = process(2 * m, buf_a, st, abase, pend, s_lo)
                start_fetch(abase, 2 * m + 2, buf_a, sem_a)
                wait_fetch(buf_b, sem_b)
                st = process(2 * m + 1, buf_b, st, abase, pend, s_lo)
                return st

            lax.fori_loop(0, n_pairs, pair_body,
                          (s_lo, p0, zeros, zeros, zeros, zeros))
            wait_fetch(buf_a, sem_a)
            pltpu.async_copy(
                outbuf, seg_hbm.at[pl.ds(seg0 + s_lo, _OUTC)], sem_a).wait()
            return jnp.int32(0)

        lax.fori_loop(0, segs_w // _OUTC, tile_body, jnp.int32(0))

    return seg_sum


@functools.lru_cache(maxsize=None)
def _make_norm(n_clusters: int, n_variants: int):
    vb = 256
    assert n_variants % vb == 0

    def body(lib_ref, seg_ref, out_ref):
        libn = lib_ref[...].reshape(n_clusters, 1, 1) / jnp.float32(1e6)
        v = seg_ref[...] / libn
        mean = jnp.mean(v, axis=0, keepdims=True)
        dlt = v - mean
        var = jnp.sum(dlt * dlt, axis=0, keepdims=True) * (
            1.0 / (n_clusters - 1))
        rel = dlt / (jnp.sqrt(var) + jnp.float32(1e-5))
        out_ref[...] = jnp.concatenate([v, rel], axis=-1)

    return pl.pallas_call(
        body,
        grid=(n_variants // vb,),
        in_specs=[
            pl.BlockSpec((n_clusters, 1), lambda i: (0, 0)),
            pl.BlockSpec((n_clusters, vb, _D), lambda i: (0, i, 0)),
        ],
        out_specs=pl.BlockSpec((n_clusters, vb, 2 * _D), lambda i: (0, i, 0)),
        out_shape=jax.ShapeDtypeStruct(
            (n_clusters, n_variants, 2 * _D), jnp.float32),
    )


def kernel(cut_embedding, local_clusterxvariant_indptr, n_variants,
           n_clusters, cluster_cut_lib):
    nc = cluster_cut_lib.shape[0]
    n_seg = local_clusterxvariant_indptr.shape[0] - 1
    nv = n_seg // nc
    n_rows = cut_embedding.shape[0]

    indptr = local_clusterxvariant_indptr.astype(jnp.int32)
    indptr = jnp.concatenate(
        [indptr, jnp.broadcast_to(indptr[-1], (16,))])

    seg = _make_seg_sum(n_rows, n_seg)(cut_embedding, indptr)
    segr = seg.reshape(nc, nv, _D)
    lib2 = cluster_cut_lib.astype(jnp.float32).reshape(nc, 1)
    return _make_norm(nc, nv)(lib2, segr)


# speedup vs baseline: 1.9755x; 1.0810x over previous
import functools

import jax
import jax.numpy as jnp
from jax import lax
from jax.experimental import pallas as pl
from jax.experimental.pallas import tpu as pltpu
from jax.experimental.pallas import tpu_sc as plsc

_D = 64
_CHUNK = 360
_OUTC = 256


@functools.lru_cache(maxsize=None)
def _make_seg_sum(n_rows: int, n_seg: int):
    info = plsc.get_sparse_core_info()
    nw = info.num_cores * info.num_subcores
    segs_w = n_seg // nw
    assert segs_w * nw == n_seg, (n_seg, nw)
    mesh = plsc.VectorSubcoreMesh(core_axis_name="c", subcore_axis_name="s")

    @functools.partial(
        pl.kernel,
        mesh=mesh,
        compiler_params=pltpu.CompilerParams(needs_layout_passes=False),
        out_type=jax.ShapeDtypeStruct((n_seg, _D), jnp.float32),
        scratch_types=[
            pltpu.VMEM((segs_w + 16,), jnp.int32),
            pltpu.VMEM((_CHUNK, _D), jnp.float32),
            pltpu.VMEM((_CHUNK, _D), jnp.float32),
            pltpu.VMEM((_OUTC, _D), jnp.float32),
            pltpu.SemaphoreType.DMA,
            pltpu.SemaphoreType.DMA,
        ],
    )
    def seg_sum(emb_hbm, indptr_hbm, seg_hbm, idx_v, buf_a, buf_b, outbuf,
                sem_a, sem_b):
        wid = lax.axis_index("s") * info.num_cores + lax.axis_index("c")
        seg0 = pl.multiple_of(wid * segs_w, 8)
        pltpu.async_copy(
            indptr_hbm.at[pl.ds(seg0, segs_w + 16)], idx_v, sem_a).wait()

        def ip(i):
            return idx_v[pl.ds(i, 16)][0]

        zeros = jnp.zeros((16,), jnp.float32)

        def chunk_base(abase, k):
            return pl.multiple_of(
                jnp.minimum(abase + k * _CHUNK,
                            jnp.int32(n_rows - _CHUNK)), 8)

        def start_fetch(abase, k, buf, sem):
            pltpu.async_copy(
                emb_hbm.at[pl.ds(chunk_base(abase, k), _CHUNK)], buf, sem)

        def wait_fetch(buf, sem):
            pltpu.make_async_copy(
                emb_hbm.at[pl.ds(0, _CHUNK)], buf, sem).wait()

        def process(k, buf, st, abase, pend, s_lo):
            seg, p, a0, a1, a2, a3 = st
            base = chunk_base(abase, k)
            hi = jnp.minimum(abase + (k + 1) * _CHUNK, pend)

            def span_add(lo, up, accs):
                @plsc.parallel_loop(lo, up, unroll=4, carry=accs)
                def out(r, accs):
                    off = r - base
                    return (accs[0] + buf[off, pl.ds(0, 16)],
                            accs[1] + buf[off, pl.ds(16, 16)],
                            accs[2] + buf[off, pl.ds(32, 16)],
                            accs[3] + buf[off, pl.ds(48, 16)])

                return out

            def cnt_body(b, acc):
                vals = idx_v[pl.ds(s_lo + 1 + b * 16, 16)]
                return acc + plsc.all_reduce_population_count(vals <= hi)[0]

            cnt = lax.fori_loop(0, _OUTC // 16, cnt_body, jnp.int32(0))
            nb = cnt - (seg - s_lo)

            def close_body(j, st):
                seg, p, a0, a1, a2, a3 = st
                e = ip(seg + 1)
                a0, a1, a2, a3 = span_add(p, e, (a0, a1, a2, a3))
                lseg = seg - s_lo
                outbuf[lseg, pl.ds(0, 16)] = a0
                outbuf[lseg, pl.ds(16, 16)] = a1
                outbuf[lseg, pl.ds(32, 16)] = a2
                outbuf[lseg, pl.ds(48, 16)] = a3
                return (seg + 1, e, zeros, zeros, zeros, zeros)

            st = lax.fori_loop(0, nb, close_body, (seg, p, a0, a1, a2, a3))
            seg, p, a0, a1, a2, a3 = st
            a0, a1, a2, a3 = span_add(p, hi, (a0, a1, a2, a3))
            return (seg, hi, a0, a1, a2, a3)

        def tile_body(f, _):
            s_lo = f * _OUTC
            p0 = ip(s_lo)
            pend = ip(s_lo + _OUTC)
            abase = p0 & jnp.int32(~7)
            n_chunks = lax.div(pend - abase, jnp.int32(_CHUNK)) + 1
            n_pairs = lax.div(n_chunks + 1, jnp.int32(2))

            start_fetch(abase, jnp.int32(0), buf_a, sem_a)

            def pair_body(m, st):
                start_fetch(abase, 2 * m + 1, buf_b, sem_b)
                wait_fetch(buf_a, sem_a)
                st = process(2 * m, buf_a, st, abase, pend, s_lo)
                start_fetch(abase, 2 * m + 2, buf_a, sem_a)
                wait_fetch(buf_b, sem_b)
                st = process(2 * m + 1, buf_b, st, abase, pend, s_lo)
                return st

            lax.fori_loop(0, n_pairs, pair_body,
                          (s_lo, p0, zeros, zeros, zeros, zeros))
            wait_fetch(buf_a, sem_a)
            pltpu.async_copy(
                outbuf, seg_hbm.at[pl.ds(seg0 + s_lo, _OUTC)], sem_a).wait()
            return jnp.int32(0)

        lax.fori_loop(0, segs_w // _OUTC, tile_body, jnp.int32(0))

    return seg_sum


@functools.lru_cache(maxsize=None)
def _make_norm(n_clusters: int, n_variants: int):
    vb = 256
    assert n_variants % vb == 0

    def body(lib_ref, seg_ref, out_ref):
        libn = lib_ref[...].reshape(n_clusters, 1, 1) / jnp.float32(1e6)
        v = seg_ref[...] / libn
        mean = jnp.mean(v, axis=0, keepdims=True)
        dlt = v - mean
        var = jnp.sum(dlt * dlt, axis=0, keepdims=True) * (
            1.0 / (n_clusters - 1))
        rel = dlt / (jnp.sqrt(var) + jnp.float32(1e-5))
        out_ref[...] = jnp.concatenate([v, rel], axis=-1)

    return pl.pallas_call(
        body,
        grid=(n_variants // vb,),
        in_specs=[
            pl.BlockSpec((n_clusters, 1), lambda i: (0, 0)),
            pl.BlockSpec((n_clusters, vb, _D), lambda i: (0, i, 0)),
        ],
        out_specs=pl.BlockSpec((n_clusters, vb, 2 * _D), lambda i: (0, i, 0)),
        out_shape=jax.ShapeDtypeStruct(
            (n_clusters, n_variants, 2 * _D), jnp.float32),
    )


def kernel(cut_embedding, local_clusterxvariant_indptr, n_variants,
           n_clusters, cluster_cut_lib):
    nc = cluster_cut_lib.shape[0]
    n_seg = local_clusterxvariant_indptr.shape[0] - 1
    nv = n_seg // nc
    n_rows = cut_embedding.shape[0]

    indptr = local_clusterxvariant_indptr.astype(jnp.int32)
    indptr = jnp.concatenate(
        [indptr, jnp.broadcast_to(indptr[-1], (16,))])

    seg = _make_seg_sum(n_rows, n_seg)(cut_embedding, indptr)
    segr = seg.reshape(nc, nv, _D)
    lib2 = cluster_cut_lib.astype(jnp.float32).reshape(nc, 1)
    return _make_norm(nc, nv)(lib2, segr)


# speedup vs baseline: 1.9909x; 1.0078x over previous
import functools

import jax
import jax.numpy as jnp
from jax import lax
from jax.experimental import pallas as pl
from jax.experimental.pallas import tpu as pltpu
from jax.experimental.pallas import tpu_sc as plsc

_D = 64
_CHUNK = 256
_OUTC = 256


@functools.lru_cache(maxsize=None)
def _make_seg_sum(n_rows: int, n_seg: int):
    info = plsc.get_sparse_core_info()
    nw = info.num_cores * info.num_subcores
    segs_w = n_seg // nw
    assert segs_w * nw == n_seg, (n_seg, nw)
    mesh = plsc.VectorSubcoreMesh(core_axis_name="c", subcore_axis_name="s")

    @functools.partial(
        pl.kernel,
        mesh=mesh,
        compiler_params=pltpu.CompilerParams(needs_layout_passes=False),
        out_type=jax.ShapeDtypeStruct((n_seg, _D), jnp.float32),
        scratch_types=[
            pltpu.VMEM((segs_w + 16,), jnp.int32),
            pltpu.VMEM((_CHUNK, _D), jnp.float32),
            pltpu.VMEM((_CHUNK, _D), jnp.float32),
            pltpu.VMEM((_OUTC, _D), jnp.float32),
            pltpu.SemaphoreType.DMA,
            pltpu.SemaphoreType.DMA,
        ],
    )
    def seg_sum(emb_hbm, indptr_hbm, seg_hbm, idx_v, buf_a, buf_b, outbuf,
                sem_a, sem_b):
        wid = lax.axis_index("s") * info.num_cores + lax.axis_index("c")
        seg0 = pl.multiple_of(wid * segs_w, 8)
        pltpu.async_copy(
            indptr_hbm.at[pl.ds(seg0, segs_w + 16)], idx_v, sem_a).wait()

        def ip(i):
            return idx_v[pl.ds(i, 16)][0]

        zeros = jnp.zeros((16,), jnp.float32)

        def chunk_base(abase, k):
            return pl.multiple_of(
                jnp.minimum(abase + k * _CHUNK,
                            jnp.int32(n_rows - _CHUNK)), 8)

        def start_fetch(abase, k, buf, sem):
            pltpu.async_copy(
                emb_hbm.at[pl.ds(chunk_base(abase, k), _CHUNK)], buf, sem)

        def wait_fetch(buf, sem):
            pltpu.make_async_copy(
                emb_hbm.at[pl.ds(0, _CHUNK)], buf, sem).wait()

        def process(k, buf, st, abase, pend, s_lo):
            seg, p, a0, a1, a2, a3 = st
            base = chunk_base(abase, k)
            hi = jnp.minimum(abase + (k + 1) * _CHUNK, pend)

            def span_add(lo, up, accs):
                @plsc.parallel_loop(lo, up, unroll=4, carry=accs)
                def out(r, accs):
                    off = r - base
                    return (accs[0] + buf[off, pl.ds(0, 16)],
                            accs[1] + buf[off, pl.ds(16, 16)],
                            accs[2] + buf[off, pl.ds(32, 16)],
                            accs[3] + buf[off, pl.ds(48, 16)])

                return out

            def cnt_body(b, acc):
                vals = idx_v[pl.ds(s_lo + 1 + b * 16, 16)]
                return acc + plsc.all_reduce_population_count(vals <= hi)[0]

            cnt = lax.fori_loop(0, _OUTC // 16, cnt_body, jnp.int32(0))
            nb = cnt - (seg - s_lo)

            def close_body(j, st):
                seg, p, a0, a1, a2, a3 = st
                e = ip(seg + 1)
                a0, a1, a2, a3 = span_add(p, e, (a0, a1, a2, a3))
                lseg = seg - s_lo
                outbuf[lseg, pl.ds(0, 16)] = a0
                outbuf[lseg, pl.ds(16, 16)] = a1
                outbuf[lseg, pl.ds(32, 16)] = a2
                outbuf[lseg, pl.ds(48, 16)] = a3
                return (seg + 1, e, zeros, zeros, zeros, zeros)

            st = lax.fori_loop(0, nb, close_body, (seg, p, a0, a1, a2, a3))
            seg, p, a0, a1, a2, a3 = st
            a0, a1, a2, a3 = span_add(p, hi, (a0, a1, a2, a3))
            return (seg, hi, a0, a1, a2, a3)

        def tile_body(f, _):
            s_lo = f * _OUTC
            p0 = ip(s_lo)
            pend = ip(s_lo + _OUTC)
            abase = p0 & jnp.int32(~7)
            n_chunks = lax.div(pend - abase, jnp.int32(_CHUNK)) + 1
            n_pairs = lax.div(n_chunks + 1, jnp.int32(2))

            start_fetch(abase, jnp.int32(0), buf_a, sem_a)

            def pair_body(m, st):
                start_fetch(abase, 2 * m + 1, buf_b, sem_b)
                wait_fetch(buf_a, sem_a)
                st = process(2 * m, buf_a, st, abase, pend, s_lo)
                start_fetch(abase, 2 * m + 2, buf_a, sem_a)
                wait_fetch(buf_b, sem_b)
                st = process(2 * m + 1, buf_b, st, abase, pend, s_lo)
                return st

            lax.fori_loop(0, n_pairs, pair_body,
                          (s_lo, p0, zeros, zeros, zeros, zeros))
            wait_fetch(buf_a, sem_a)
            pltpu.async_copy(
                outbuf, seg_hbm.at[pl.ds(seg0 + s_lo, _OUTC)], sem_a).wait()
            return jnp.int32(0)

        lax.fori_loop(0, segs_w // _OUTC, tile_body, jnp.int32(0))

    return seg_sum


@functools.lru_cache(maxsize=None)
def _make_norm(n_clusters: int, n_variants: int):
    vb = 256
    assert n_variants % vb == 0

    def body(lib_ref, seg_ref, out_ref):
        libn = lib_ref[...].reshape(n_clusters, 1, 1) / jnp.float32(1e6)
        v = seg_ref[...] / libn
        mean = jnp.mean(v, axis=0, keepdims=True)
        dlt = v - mean
        var = jnp.sum(dlt * dlt, axis=0, keepdims=True) * (
            1.0 / (n_clusters - 1))
        rel = dlt / (jnp.sqrt(var) + jnp.float32(1e-5))
        out_ref[...] = jnp.concatenate([v, rel], axis=-1)

    return pl.pallas_call(
        body,
        grid=(n_variants // vb,),
        in_specs=[
            pl.BlockSpec((n_clusters, 1), lambda i: (0, 0)),
            pl.BlockSpec((n_clusters, vb, _D), lambda i: (0, i, 0)),
        ],
        out_specs=pl.BlockSpec((n_clusters, vb, 2 * _D), lambda i: (0, i, 0)),
        out_shape=jax.ShapeDtypeStruct(
            (n_clusters, n_variants, 2 * _D), jnp.float32),
    )


def kernel(cut_embedding, local_clusterxvariant_indptr, n_variants,
           n_clusters, cluster_cut_lib):
    nc = cluster_cut_lib.shape[0]
    n_seg = local_clusterxvariant_indptr.shape[0] - 1
    nv = n_seg // nc
    n_rows = cut_embedding.shape[0]

    indptr = local_clusterxvariant_indptr.astype(jnp.int32)
    indptr = jnp.concatenate(
        [indptr, jnp.broadcast_to(indptr[-1], (16,))])

    seg = _make_seg_sum(n_rows, n_seg)(cut_embedding, indptr)
    segr = seg.reshape(nc, nv, _D)
    lib2 = cluster_cut_lib.astype(jnp.float32).reshape(nc, 1)
    return _make_norm(nc, nv)(lib2, segr)
